# baseline (speedup 1.0000x reference)
import jax
import jax.numpy as jnp
from jax import lax
from jax.experimental import pallas as pl
from jax.experimental.pallas import tpu as pltpu

NUM_CLASSES = 1000
SPLIT = 896
CHUNK = 64
NBUF = 4


def _onehot_body(tgt_ref, out_ref, sa_ref, sb_ref, sem_ref):
    b, s = tgt_ref.shape
    n_steps = b // CHUNK

    def _copy_a(i, buf):
        return pltpu.make_async_copy(
            sa_ref.at[buf],
            out_ref.at[pl.ds(i * CHUNK, CHUNK), :, pl.ds(0, SPLIT)],
            sem_ref.at[buf, 0],
        )

    def _copy_b(i, buf):
        return pltpu.make_async_copy(
            sb_ref.at[buf],
            out_ref.at[pl.ds(i * CHUNK, CHUNK), :,
                       pl.ds(SPLIT, NUM_CLASSES - SPLIT)],
            sem_ref.at[buf, 1],
        )

    def step(i, carry):
        buf = lax.rem(i, NBUF)

        @pl.when(i >= NBUF)
        def _():
            _copy_a(i - NBUF, buf).wait()
            _copy_b(i - NBUF, buf).wait()

        tgt = tgt_ref[pl.ds(i * CHUNK, CHUNK), :]
        iota_a = lax.broadcasted_iota(jnp.int32, (CHUNK, s, SPLIT), 2)
        iota_b = lax.broadcasted_iota(
            jnp.int32, (CHUNK, s, NUM_CLASSES - SPLIT), 2) + SPLIT
        sa_ref[buf] = (iota_a == tgt[:, :, None]).astype(jnp.float32)
        sb_ref[buf] = (iota_b == tgt[:, :, None]).astype(jnp.float32)
        _copy_a(i, buf).start()
        _copy_b(i, buf).start()
        return carry

    lax.fori_loop(0, n_steps, step, 0)
    for j in range(NBUF):
        i = n_steps - NBUF + j
        _copy_a(i, i % NBUF).wait()
        _copy_b(i, i % NBUF).wait()


def kernel(target):
    b, s = target.shape
    return pl.pallas_call(
        _onehot_body,
        in_specs=[pl.BlockSpec(memory_space=pltpu.MemorySpace.VMEM)],
        out_specs=pl.BlockSpec(memory_space=pltpu.MemorySpace.HBM),
        out_shape=jax.ShapeDtypeStruct((b, s, NUM_CLASSES), jnp.float32),
        scratch_shapes=[
            pltpu.VMEM((NBUF, CHUNK, s, SPLIT), jnp.float32),
            pltpu.VMEM((NBUF, CHUNK, s, NUM_CLASSES - SPLIT), jnp.float32),
            pltpu.SemaphoreType.DMA((NBUF, 2)),
        ],
    )(target)
